# baseline (speedup 1.0000x reference)
import functools

import jax
import jax.numpy as jnp
from jax import lax
from jax.experimental import pallas as pl
from jax.experimental.pallas import tpu as pltpu
from jax.experimental.pallas import tpu_sc as plsc

VOCAB = 1000
EMBED = 64
BATCH = 4096
FIELDS = 26
FIELDS_PAD = 32
NUM_WORKERS = 32
ROWS_PER_W = BATCH // NUM_WORKERS
N_GROUPS = ROWS_PER_W * FIELDS_PAD // 128
NBUF = 6
G_AHEAD = 3


def _sc_embedding_gather(table_pad, idx_op):
    mesh = plsc.VectorSubcoreMesh(core_axis_name="c", subcore_axis_name="s")

    @functools.partial(
        pl.kernel,
        mesh=mesh,
        out_type=jax.ShapeDtypeStruct((BATCH, FIELDS_PAD, 128), jnp.float32),
        compiler_params=pltpu.CompilerParams(use_tc_tiling_on_sc=False),
        scratch_types=[
            pltpu.VMEM((N_GROUPS, 128), jnp.int32),
            pltpu.VMEM((NBUF, 128, 128), jnp.float32),
            pltpu.SemaphoreType.DMA,
            pltpu.SemaphoreType.DMA,
        ],
    )
    def k(table_hbm, idx_hbm, out_hbm, idx_v, rows_v, gsem, osem):
        wid = lax.axis_index("s") * 2 + lax.axis_index("c")
        row0 = wid * ROWS_PER_W

        pltpu.sync_copy(idx_hbm.at[pl.ds(wid * N_GROUPS, N_GROUPS)], idx_v)

        def gather(g, b):
            return pltpu.make_async_copy(
                table_hbm.at[idx_v.at[g]], rows_v.at[b], gsem)

        def out_copy(g, b, j):
            return pltpu.make_async_copy(
                rows_v.at[b, pl.ds(j * FIELDS_PAD, FIELDS_PAD)],
                out_hbm.at[row0 + 4 * g + j],
                osem)

        for g in range(G_AHEAD):
            gather(g, g).start()

        def body(g, _):
            b = lax.rem(g, NBUF)
            ng = g + G_AHEAD
            fire = ng < N_GROUPS

            @pl.when(jnp.logical_and(g >= G_AHEAD, fire))
            def _():
                for j in range(4):
                    out_copy(g, b, j).wait()

            @pl.when(fire)
            def _():
                gather(ng, lax.rem(ng, NBUF)).start()

            gather(g, b).wait()
            for j in range(4):
                out_copy(g, b, j).start()
            return ()

        lax.fori_loop(0, N_GROUPS, body, (), unroll=False)

        for i in range(NBUF):
            g = N_GROUPS - NBUF + i
            for j in range(4):
                out_copy(g, g % NBUF, j).wait()

    return k(table_pad, idx_op)


def kernel(inputs, table):
    idx_op = jnp.pad(inputs, ((0, 0), (0, FIELDS_PAD - FIELDS))).reshape(
        BATCH * FIELDS_PAD // 128, 128)
    table_pad = jnp.pad(table, ((0, 0), (0, 128 - EMBED)))
    out = _sc_embedding_gather(table_pad, idx_op)
    return out[:, :FIELDS, :EMBED]


# speedup vs baseline: 1.0004x; 1.0004x over previous
import functools

import jax
import jax.numpy as jnp
from jax import lax
from jax.experimental import pallas as pl
from jax.experimental.pallas import tpu as pltpu
from jax.experimental.pallas import tpu_sc as plsc

VOCAB = 1000
EMBED = 64
BATCH = 4096
FIELDS = 26
FIELDS_PAD = 32
NUM_WORKERS = 32
ROWS_PER_W = BATCH // NUM_WORKERS
N_GROUPS = ROWS_PER_W * FIELDS_PAD // 128
NBUF = 6
G_AHEAD = 3


def _sc_embedding_gather(table_pad, idx_op):
    mesh = plsc.VectorSubcoreMesh(core_axis_name="c", subcore_axis_name="s")

    @functools.partial(
        pl.kernel,
        mesh=mesh,
        out_type=jax.ShapeDtypeStruct((BATCH * FIELDS_PAD, 128), jnp.float32),
        compiler_params=pltpu.CompilerParams(use_tc_tiling_on_sc=False),
        scratch_types=[
            pltpu.VMEM((N_GROUPS, 128), jnp.int32),
            pltpu.VMEM((NBUF, 128, 128), jnp.float32),
            pltpu.SemaphoreType.DMA,
            pltpu.SemaphoreType.DMA,
        ],
    )
    def k(table_hbm, idx_hbm, out_hbm, idx_v, rows_v, gsem, osem):
        wid = lax.axis_index("s") * 2 + lax.axis_index("c")
        row0 = wid * ROWS_PER_W

        pltpu.sync_copy(idx_hbm.at[pl.ds(wid * N_GROUPS, N_GROUPS)], idx_v)

        def gather(g, b):
            return pltpu.make_async_copy(
                table_hbm.at[idx_v.at[g]], rows_v.at[b], gsem)

        def out_copy(g, b):
            return pltpu.make_async_copy(
                rows_v.at[b],
                out_hbm.at[pl.ds((row0 + 4 * g) * FIELDS_PAD, 128)],
                osem)

        for g in range(G_AHEAD):
            gather(g, g).start()

        def body(g, _):
            b = lax.rem(g, NBUF)
            ng = g + G_AHEAD
            fire = ng < N_GROUPS

            @pl.when(jnp.logical_and(g >= G_AHEAD, fire))
            def _():
                out_copy(g, b).wait()

            @pl.when(fire)
            def _():
                gather(ng, lax.rem(ng, NBUF)).start()

            gather(g, b).wait()
            out_copy(g, b).start()
            return ()

        lax.fori_loop(0, N_GROUPS, body, (), unroll=False)

        for i in range(NBUF):
            g = N_GROUPS - NBUF + i
            out_copy(g, g % NBUF).wait()

    return k(table_pad, idx_op)


def kernel(inputs, table):
    idx_op = jnp.pad(inputs, ((0, 0), (0, FIELDS_PAD - FIELDS))).reshape(
        BATCH * FIELDS_PAD // 128, 128)
    table_pad = jnp.pad(table, ((0, 0), (0, 128 - EMBED)))
    out = _sc_embedding_gather(table_pad, idx_op)
    return out.reshape(BATCH, FIELDS_PAD, 128)[:, :FIELDS, :EMBED]


# speedup vs baseline: 8.9853x; 8.9814x over previous
import functools

import jax
import jax.numpy as jnp
from jax import lax
from jax.experimental import pallas as pl
from jax.experimental.pallas import tpu as pltpu
from jax.experimental.pallas import tpu_sc as plsc

VOCAB = 1000
EMBED = 64
BATCH = 4096
FIELDS = 26
FIELDS_PAD = 32
NUM_WORKERS = 32
ROWS_PER_W = BATCH // NUM_WORKERS
N_GROUPS = ROWS_PER_W * FIELDS_PAD // 128
NBUF = 6
G_AHEAD = 3


def _sc_embedding_gather(table_pad, idx_op):
    mesh = plsc.VectorSubcoreMesh(core_axis_name="c", subcore_axis_name="s")

    @functools.partial(
        pl.kernel,
        mesh=mesh,
        out_type=jax.ShapeDtypeStruct((BATCH * FIELDS_PAD, 128), jnp.float32),
        compiler_params=pltpu.CompilerParams(use_tc_tiling_on_sc=False),
        scratch_types=[
            pltpu.VMEM((N_GROUPS, 128), jnp.int32),
            pltpu.VMEM((NBUF, 128, 128), jnp.float32),
            pltpu.SemaphoreType.DMA,
            pltpu.SemaphoreType.DMA,
        ],
    )
    def k(table_hbm, idx_hbm, out_hbm, idx_v, rows_v, gsem, osem):
        wid = lax.axis_index("s") * 2 + lax.axis_index("c")
        row0 = wid * ROWS_PER_W

        pltpu.sync_copy(idx_hbm.at[wid], idx_v)

        def gather(g, b):
            return pltpu.make_async_copy(
                table_hbm.at[idx_v.at[g]], rows_v.at[b], gsem)

        def out_copy(g, b):
            return pltpu.make_async_copy(
                rows_v.at[b],
                out_hbm.at[pl.ds((row0 + 4 * g) * FIELDS_PAD, 128)],
                osem)

        for g in range(G_AHEAD):
            gather(g, g).start()

        def body(g, _):
            b = lax.rem(g, NBUF)
            ng = g + G_AHEAD
            fire = ng < N_GROUPS

            @pl.when(jnp.logical_and(g >= G_AHEAD, fire))
            def _():
                out_copy(g, b).wait()

            @pl.when(fire)
            def _():
                gather(ng, lax.rem(ng, NBUF)).start()

            gather(g, b).wait()
            out_copy(g, b).start()
            return ()

        lax.fori_loop(0, N_GROUPS, body, (), unroll=False)

        for i in range(NBUF):
            g = N_GROUPS - NBUF + i
            out_copy(g, g % NBUF).wait()

    return k(table_pad, idx_op)


def kernel(inputs, table):
    pad_vals = (jnp.arange(BATCH, dtype=jnp.int32)[:, None] * 7
                + jnp.arange(FIELDS_PAD - FIELDS, dtype=jnp.int32)[None, :]
                * 131) % VOCAB
    idx_op = jnp.concatenate([inputs, pad_vals], axis=1).reshape(
        NUM_WORKERS, N_GROUPS, 128)
    table_pad = jnp.pad(table, ((0, 0), (0, 128 - EMBED)))
    out = _sc_embedding_gather(table_pad, idx_op)
    return out.reshape(BATCH, FIELDS_PAD, 128)[:, :FIELDS, :EMBED]
